# baseline (speedup 1.0000x reference)
import functools

import jax
import jax.numpy as jnp
from jax import lax
from jax.experimental import pallas as pl
from jax.experimental.pallas import tpu as pltpu
from jax.experimental.pallas import tpu_sc as plsc

B, C, M, W = 16, 4, 32, 16
D_MODEL = 256
D_K = 64
N = B * C * M
TOK = N * W

NC, NS = 2, 16
NW = NC * NS
TPW = TOK // NW
CHUNK = 128
CPW = TPW // CHUNK
NBUF = 3


def _sc_gather(table, ids):
  mesh = plsc.VectorSubcoreMesh(core_axis_name="c", subcore_axis_name="s")

  @functools.partial(
      pl.kernel,
      mesh=mesh,
      out_type=jax.ShapeDtypeStruct((TOK, D_MODEL), jnp.float32),
      scratch_types=[
          pltpu.VMEM((TPW,), jnp.int32),
          pltpu.VMEM((NBUF, CHUNK, D_MODEL), jnp.float32),
          pltpu.SemaphoreType.DMA,
          pltpu.SemaphoreType.DMA,
          pltpu.SemaphoreType.DMA,
          pltpu.SemaphoreType.DMA,
          pltpu.SemaphoreType.DMA,
          pltpu.SemaphoreType.DMA,
      ],
  )
  def gather_kernel(table_hbm, ids_hbm, out_hbm, idx_v, rows_v,
                    g0, g1, g2, w0, w1, w2):
    wid = lax.axis_index("s") * NC + lax.axis_index("c")
    base = wid * TPW
    pltpu.sync_copy(ids_hbm.at[pl.ds(base, TPW)], idx_v)
    gsems = [g0, g1, g2]
    wsems = [w0, w1, w2]

    def gstart(g):
      b = g % NBUF
      return pltpu.async_copy(
          table_hbm.at[idx_v.at[pl.ds(g * CHUNK, CHUNK)]],
          rows_v.at[b], gsems[b])

    gh = [None] * CPW
    wh = [None] * CPW
    for g in range(min(NBUF, CPW)):
      gh[g] = gstart(g)
    for g in range(CPW):
      b = g % NBUF
      gh[g].wait()
      wh[g] = pltpu.async_copy(
          rows_v.at[b], out_hbm.at[pl.ds(base + g * CHUNK, CHUNK)], wsems[b])
      if g + NBUF < CPW:
        wh[g].wait()
        gh[g + NBUF] = gstart(g + NBUF)
    for g in range(max(0, CPW - NBUF), CPW):
      wh[g].wait()

  return gather_kernel(table, ids)


def _tc_body(bags_ref, h_ref, wp_ref, bp_ref, wq_ref, bq_ref, v_ref,
             lens_ref, szs_ref, out_ref):
  bags = bags_ref[...]
  bags_bf = bags.astype(jnp.bfloat16)
  pre = jnp.dot(bags_bf, wp_ref[...].astype(jnp.bfloat16),
                preferred_element_type=jnp.float32) + bp_ref[...]
  q = jnp.dot(h_ref[0], wq_ref[...],
              preferred_element_type=jnp.float32) + bq_ref[...]
  e = jnp.dot(jnp.tanh(pre + q), v_ref[...],
              preferred_element_type=jnp.float32)
  wpos = lax.broadcasted_iota(jnp.int32, (W * N // B, 1), 0) % W
  wmask = (wpos.astype(jnp.float32) < lens_ref[...]).astype(jnp.float32)
  p = jnp.exp(e) * wmask
  nodes_per = N // B
  t_of = lax.broadcasted_iota(jnp.int32, (W * nodes_per, nodes_per), 0) // W
  n_of = lax.broadcasted_iota(jnp.int32, (W * nodes_per, nodes_per), 1)
  S = (t_of == n_of).astype(jnp.float32)
  n2 = lax.broadcasted_iota(jnp.int32, (nodes_per, W * nodes_per), 0)
  t2 = lax.broadcasted_iota(jnp.int32, (nodes_per, W * nodes_per), 1) // W
  S2 = (n2 == t2).astype(jnp.float32)
  nsum = jnp.dot(S2, p, preferred_element_type=jnp.float32)
  denom = jnp.dot(S, nsum, preferred_element_type=jnp.float32)
  attn = p / denom
  weighted = (bags * attn).astype(jnp.bfloat16)
  ctx = jnp.dot(S2.astype(jnp.bfloat16), weighted,
                preferred_element_type=jnp.float32)
  mpos = lax.broadcasted_iota(jnp.int32, (nodes_per, 1), 0) % M
  nmask = (mpos.astype(jnp.float32) < szs_ref[...]).astype(jnp.float32)
  out_ref[...] = ctx * nmask


def _tc_attention(bags, hidden, W_pre, b_pre, W_q, b_q, v_att,
                  lens_tok, sizes_node):
  nodes_per = N // B
  return pl.pallas_call(
      _tc_body,
      grid=(B,),
      in_specs=[
          pl.BlockSpec((W * nodes_per, D_MODEL), lambda i: (i, 0)),
          pl.BlockSpec((1, 1, D_MODEL), lambda i: (i, 0, 0)),
          pl.BlockSpec((D_MODEL, D_K), lambda i: (0, 0)),
          pl.BlockSpec((1, D_K), lambda i: (0, 0)),
          pl.BlockSpec((D_MODEL, D_K), lambda i: (0, 0)),
          pl.BlockSpec((1, D_K), lambda i: (0, 0)),
          pl.BlockSpec((D_K, 1), lambda i: (0, 0)),
          pl.BlockSpec((W * nodes_per, 1), lambda i: (i, 0)),
          pl.BlockSpec((nodes_per, 1), lambda i: (i, 0)),
      ],
      out_specs=pl.BlockSpec((nodes_per, D_MODEL), lambda i: (i, 0)),
      out_shape=jax.ShapeDtypeStruct((N, D_MODEL), jnp.float32),
  )(bags, hidden, W_pre, b_pre, W_q, b_q, v_att, lens_tok, sizes_node)


def kernel(con_hidden, emb_table, W_pre, b_pre, W_q, b_q, v_att,
           token_ids, node_lengths, node_sizes):
  hidden = jnp.concatenate([con_hidden[0], con_hidden[1]], axis=1)
  ids = token_ids.reshape(TOK)
  bags = _sc_gather(emb_table, ids)
  lens_tok = jnp.repeat(node_lengths, W).astype(jnp.float32).reshape(TOK, 1)
  sizes_node = jnp.repeat(node_sizes, M).astype(jnp.float32).reshape(N, 1)
  ctx = _tc_attention(bags, hidden.reshape(B, 1, D_MODEL), W_pre,
                      b_pre.reshape(1, D_K),
                      W_q, b_q.reshape(1, D_K), v_att.reshape(D_K, 1),
                      lens_tok, sizes_node)
  return (ctx.reshape(B, C, M, D_MODEL), hidden)


# speedup vs baseline: 1.1844x; 1.1844x over previous
import functools

import jax
import jax.numpy as jnp
from jax import lax
from jax.experimental import pallas as pl
from jax.experimental.pallas import tpu as pltpu
from jax.experimental.pallas import tpu_sc as plsc

B, C, M, W = 16, 4, 32, 16
D_MODEL = 256
D_K = 64
N = B * C * M
TOK = N * W

NC, NS = 2, 16
NW = NC * NS
TPW = TOK // NW
CHUNK = 128
CPW = TPW // CHUNK
NBUF = 3


def _sc_gather(table, ids):
  mesh = plsc.VectorSubcoreMesh(core_axis_name="c", subcore_axis_name="s")

  @functools.partial(
      pl.kernel,
      mesh=mesh,
      out_type=jax.ShapeDtypeStruct((TOK, D_MODEL), jnp.float32),
      scratch_types=[
          pltpu.VMEM((TPW,), jnp.int32),
          pltpu.VMEM((NBUF, CHUNK, D_MODEL), jnp.float32),
          pltpu.SemaphoreType.DMA,
          pltpu.SemaphoreType.DMA,
          pltpu.SemaphoreType.DMA,
          pltpu.SemaphoreType.DMA,
          pltpu.SemaphoreType.DMA,
          pltpu.SemaphoreType.DMA,
      ],
  )
  def gather_kernel(table_hbm, ids_hbm, out_hbm, idx_v, rows_v,
                    g0, g1, g2, w0, w1, w2):
    wid = lax.axis_index("s") * NC + lax.axis_index("c")
    base = wid * TPW
    pltpu.sync_copy(ids_hbm.at[pl.ds(base, TPW)], idx_v)
    gsems = [g0, g1, g2]
    wsems = [w0, w1, w2]

    def gstart(g):
      b = g % NBUF
      return pltpu.async_copy(
          table_hbm.at[idx_v.at[pl.ds(g * CHUNK, CHUNK)]],
          rows_v.at[b], gsems[b])

    gh = [None] * CPW
    wh = [None] * CPW
    for g in range(min(NBUF, CPW)):
      gh[g] = gstart(g)
    for g in range(CPW):
      b = g % NBUF
      gh[g].wait()
      wh[g] = pltpu.async_copy(
          rows_v.at[b], out_hbm.at[pl.ds(base + g * CHUNK, CHUNK)], wsems[b])
      if g + NBUF < CPW:
        wh[g].wait()
        gh[g + NBUF] = gstart(g + NBUF)
    for g in range(max(0, CPW - NBUF), CPW):
      wh[g].wait()

  return gather_kernel(table, ids)


NPB = N // B


def _tc_body(bags_ref, h_ref, wp_ref, bp_ref, wq_ref, bq_ref, v_ref,
             lens_ref, szs_ref, out_ref):
  flat = bags_ref[...].reshape(W * NPB, D_MODEL)
  pre = jnp.dot(flat.astype(jnp.bfloat16), wp_ref[...].astype(jnp.bfloat16),
                preferred_element_type=jnp.float32) + bp_ref[...]
  q = jnp.dot(h_ref[0], wq_ref[...],
              preferred_element_type=jnp.float32) + bq_ref[...]
  e = jnp.dot(jnp.tanh(pre + q), v_ref[...],
              preferred_element_type=jnp.float32)
  p = jnp.exp(e)
  lens = lens_ref[...]
  ps = []
  denom = None
  for w in range(W):
    pw = p[w * NPB:(w + 1) * NPB] * (lens > float(w)).astype(jnp.float32)
    ps.append(pw)
    denom = pw if w == 0 else denom + pw
  mpos = lax.broadcasted_iota(jnp.int32, (NPB, 1), 0) % M
  nmask = (mpos.astype(jnp.float32) < szs_ref[...]).astype(jnp.float32)
  scale = nmask / denom
  ctx = ps[0] * scale * bags_ref[0]
  for w in range(1, W):
    ctx = ctx + (ps[w] * scale) * bags_ref[w]
  out_ref[...] = ctx


def _tc_attention(bags_t, hidden, W_pre, b_pre, W_q, b_q, v_att,
                  lens_node, sizes_node):
  return pl.pallas_call(
      _tc_body,
      grid=(B,),
      in_specs=[
          pl.BlockSpec((W, NPB, D_MODEL), lambda i: (0, i, 0)),
          pl.BlockSpec((1, 1, D_MODEL), lambda i: (i, 0, 0)),
          pl.BlockSpec((D_MODEL, D_K), lambda i: (0, 0)),
          pl.BlockSpec((1, D_K), lambda i: (0, 0)),
          pl.BlockSpec((D_MODEL, D_K), lambda i: (0, 0)),
          pl.BlockSpec((1, D_K), lambda i: (0, 0)),
          pl.BlockSpec((D_K, 1), lambda i: (0, 0)),
          pl.BlockSpec((NPB, 1), lambda i: (i, 0)),
          pl.BlockSpec((NPB, 1), lambda i: (i, 0)),
      ],
      out_specs=pl.BlockSpec((NPB, D_MODEL), lambda i: (i, 0)),
      out_shape=jax.ShapeDtypeStruct((N, D_MODEL), jnp.float32),
  )(bags_t, hidden, W_pre, b_pre, W_q, b_q, v_att, lens_node, sizes_node)


def kernel(con_hidden, emb_table, W_pre, b_pre, W_q, b_q, v_att,
           token_ids, node_lengths, node_sizes):
  hidden = jnp.concatenate([con_hidden[0], con_hidden[1]], axis=1)
  ids_t = token_ids.reshape(N, W).T.reshape(TOK)
  bags_t = _sc_gather(emb_table, ids_t).reshape(W, N, D_MODEL)
  lens_node = node_lengths.astype(jnp.float32).reshape(N, 1)
  sizes_node = jnp.repeat(node_sizes, M).astype(jnp.float32).reshape(N, 1)
  ctx = _tc_attention(bags_t, hidden.reshape(B, 1, D_MODEL), W_pre,
                      b_pre.reshape(1, D_K),
                      W_q, b_q.reshape(1, D_K), v_att.reshape(D_K, 1),
                      lens_node, sizes_node)
  return (ctx.reshape(B, C, M, D_MODEL), hidden)


# speedup vs baseline: 1.2002x; 1.0133x over previous
import functools

import jax
import jax.numpy as jnp
from jax import lax
from jax.experimental import pallas as pl
from jax.experimental.pallas import tpu as pltpu
from jax.experimental.pallas import tpu_sc as plsc

B, C, M, W = 16, 4, 32, 16
D_MODEL = 256
D_K = 64
N = B * C * M
TOK = N * W

NC, NS = 2, 16
NW = NC * NS
CHUNK = 128
NBUF = 3


def _sc_gather(table, ids):
  ntok = ids.shape[0]
  tpw = ntok // NW
  cpw = tpw // CHUNK
  mesh = plsc.VectorSubcoreMesh(core_axis_name="c", subcore_axis_name="s")

  @functools.partial(
      pl.kernel,
      mesh=mesh,
      out_type=jax.ShapeDtypeStruct((ntok, D_MODEL), jnp.float32),
      scratch_types=[
          pltpu.VMEM((tpw,), jnp.int32),
          pltpu.VMEM((NBUF, CHUNK, D_MODEL), jnp.float32),
          pltpu.SemaphoreType.DMA,
          pltpu.SemaphoreType.DMA,
          pltpu.SemaphoreType.DMA,
          pltpu.SemaphoreType.DMA,
          pltpu.SemaphoreType.DMA,
          pltpu.SemaphoreType.DMA,
      ],
  )
  def gather_kernel(table_hbm, ids_hbm, out_hbm, idx_v, rows_v,
                    g0, g1, g2, w0, w1, w2):
    CPW = cpw
    wid = lax.axis_index("s") * NC + lax.axis_index("c")
    base = wid * tpw
    pltpu.sync_copy(ids_hbm.at[pl.ds(base, tpw)], idx_v)
    gsems = [g0, g1, g2]
    wsems = [w0, w1, w2]

    def gstart(g):
      b = g % NBUF
      return pltpu.async_copy(
          table_hbm.at[idx_v.at[pl.ds(g * CHUNK, CHUNK)]],
          rows_v.at[b], gsems[b])

    gh = [None] * CPW
    wh = [None] * CPW
    for g in range(min(NBUF, CPW)):
      gh[g] = gstart(g)
    for g in range(CPW):
      b = g % NBUF
      gh[g].wait()
      wh[g] = pltpu.async_copy(
          rows_v.at[b], out_hbm.at[pl.ds(base + g * CHUNK, CHUNK)], wsems[b])
      if g + NBUF < CPW:
        wh[g].wait()
        gh[g + NBUF] = gstart(g + NBUF)
    for g in range(max(0, CPW - NBUF), CPW):
      wh[g].wait()

  return gather_kernel(table, ids)


NPB = N // B


def _tc_body(bags_ref, h_ref, wp_ref, bp_ref, wq_ref, bq_ref, v_ref,
             lens_ref, szs_ref, out_ref):
  flat = bags_ref[...].reshape(W * NPB, D_MODEL)
  pre = jnp.dot(flat.astype(jnp.bfloat16), wp_ref[...].astype(jnp.bfloat16),
                preferred_element_type=jnp.float32) + bp_ref[...]
  q = jnp.dot(h_ref[0], wq_ref[...],
              preferred_element_type=jnp.float32) + bq_ref[...]
  e = jnp.dot(jnp.tanh(pre + q), v_ref[...],
              preferred_element_type=jnp.float32)
  p = jnp.exp(e)
  lens = lens_ref[...]
  ps = []
  denom = None
  for w in range(W):
    pw = p[w * NPB:(w + 1) * NPB] * (lens > float(w)).astype(jnp.float32)
    ps.append(pw)
    denom = pw if w == 0 else denom + pw
  mpos = lax.broadcasted_iota(jnp.int32, (NPB, 1), 0) % M
  nmask = (mpos.astype(jnp.float32) < szs_ref[...]).astype(jnp.float32)
  scale = nmask / denom
  ctx = ps[0] * scale * bags_ref[0]
  for w in range(1, W):
    ctx = ctx + (ps[w] * scale) * bags_ref[w]
  out_ref[...] = ctx


def _tc_attention(bags_t, hidden, W_pre, b_pre, W_q, b_q, v_att,
                  lens_node, sizes_node):
  nb = hidden.shape[0]
  return pl.pallas_call(
      _tc_body,
      grid=(nb,),
      in_specs=[
          pl.BlockSpec((W, NPB, D_MODEL), lambda i: (0, i, 0)),
          pl.BlockSpec((1, 1, D_MODEL), lambda i: (i, 0, 0)),
          pl.BlockSpec((D_MODEL, D_K), lambda i: (0, 0)),
          pl.BlockSpec((1, D_K), lambda i: (0, 0)),
          pl.BlockSpec((D_MODEL, D_K), lambda i: (0, 0)),
          pl.BlockSpec((1, D_K), lambda i: (0, 0)),
          pl.BlockSpec((D_K, 1), lambda i: (0, 0)),
          pl.BlockSpec((NPB, 1), lambda i: (i, 0)),
          pl.BlockSpec((NPB, 1), lambda i: (i, 0)),
      ],
      out_specs=pl.BlockSpec((NPB, D_MODEL), lambda i: (i, 0)),
      out_shape=jax.ShapeDtypeStruct((nb * NPB, D_MODEL), jnp.float32),
  )(bags_t, hidden, W_pre, b_pre, W_q, b_q, v_att, lens_node, sizes_node)


NSPLIT = 2


def kernel(con_hidden, emb_table, W_pre, b_pre, W_q, b_q, v_att,
           token_ids, node_lengths, node_sizes):
  hidden = jnp.concatenate([con_hidden[0], con_hidden[1]], axis=1)
  ids2 = token_ids.reshape(N, W).T
  lens_node = node_lengths.astype(jnp.float32).reshape(N, 1)
  sizes_node = jnp.repeat(node_sizes, M).astype(jnp.float32).reshape(N, 1)
  h3 = hidden.reshape(B, 1, D_MODEL)
  bp2, bq2, v2 = b_pre.reshape(1, D_K), b_q.reshape(1, D_K), v_att.reshape(D_K, 1)
  nh = N // NSPLIT
  bh = B // NSPLIT
  bags = [
      _sc_gather(emb_table, ids2[:, k * nh:(k + 1) * nh].reshape(W * nh))
      .reshape(W, nh, D_MODEL)
      for k in range(NSPLIT)
  ]
  ctxs = [
      _tc_attention(bags[k], h3[k * bh:(k + 1) * bh], W_pre, bp2, W_q, bq2, v2,
                    lens_node[k * nh:(k + 1) * nh],
                    sizes_node[k * nh:(k + 1) * nh])
      for k in range(NSPLIT)
  ]
  ctx = jnp.concatenate(ctxs, axis=0)
  return (ctx.reshape(B, C, M, D_MODEL), hidden)
